# Initial kernel scaffold; baseline (speedup 1.0000x reference)
import jax
import jax.numpy as jnp
from jax.experimental import pallas as pl


def kernel(dense_input, sparse_input, weight_sparse, weight_dense, bias):
    raise NotImplementedError("write your pallas kernel here")


# speedup vs baseline: 1.1511x; 1.1511x over previous
import jax
import jax.numpy as jnp
from jax import lax
from jax.experimental import pallas as pl
from jax.experimental.pallas import tpu as pltpu, tpu_sc as plsc

_VOCAB = 1000012
_BATCH = 16384
_N_SPARSE = 26
_D_DENSE = 13

_NC = 2
_NS = 16
_NW = _NC * _NS
_BPW = _BATCH // _NW
_IPW = _BPW * _N_SPARSE
_NBLK = _BPW // 16


def _sc_body(dense_hbm, idx_hbm, table_hbm, consts_hbm, out_hbm,
             idx_v, vals_v, dense_v, consts_v, out_v, sem):
    wid = lax.axis_index("s") * _NC + lax.axis_index("c")
    base = wid * _BPW

    pltpu.sync_copy(idx_hbm.at[pl.ds(base * _N_SPARSE, _IPW)], idx_v)
    gat = pltpu.async_copy(table_hbm.at[idx_v], vals_v, sem)
    pltpu.sync_copy(dense_hbm.at[pl.ds(base * _D_DENSE, _BPW * _D_DENSE)],
                    dense_v)
    pltpu.sync_copy(consts_hbm, consts_v)
    gat.wait()

    cv = consts_v[...]

    def blk_body(blk, carry):
        rows = lax.iota(jnp.int32, 16) + blk * 16
        acc = jnp.full((16,), cv[_D_DENSE], jnp.float32)
        dbase = rows * _D_DENSE
        for k in range(_D_DENSE):
            acc = acc + plsc.load_gather(dense_v, [dbase + k]) * cv[k]
        vbase = rows * _N_SPARSE
        for f in range(_N_SPARSE):
            acc = acc + plsc.load_gather(vals_v, [vbase + f])
        out_v[pl.ds(blk * 16, 16)] = acc
        return carry

    lax.fori_loop(0, _NBLK, blk_body, 0)
    pltpu.sync_copy(out_v, out_hbm.at[pl.ds(base, _BPW)])


def kernel(dense_input, sparse_input, weight_sparse, weight_dense, bias):
    idx = sparse_input.astype(jnp.int32).reshape(_BATCH * _N_SPARSE)
    dense_flat = dense_input.reshape(_BATCH * _D_DENSE)
    table_flat = weight_sparse.reshape(_VOCAB)
    consts = jnp.concatenate(
        [weight_dense.reshape(_D_DENSE), bias,
         jnp.zeros((16 - _D_DENSE - 1,), jnp.float32)])
    mesh = plsc.VectorSubcoreMesh(core_axis_name="c", subcore_axis_name="s")
    run = pl.kernel(
        _sc_body,
        out_type=jax.ShapeDtypeStruct((_BATCH,), jnp.float32),
        mesh=mesh,
        compiler_params=pltpu.CompilerParams(needs_layout_passes=False),
        scratch_types=[
            pltpu.VMEM((_IPW,), jnp.int32),
            pltpu.VMEM((_IPW,), jnp.float32),
            pltpu.VMEM((_BPW * _D_DENSE,), jnp.float32),
            pltpu.VMEM((16,), jnp.float32),
            pltpu.VMEM((_BPW,), jnp.float32),
            pltpu.SemaphoreType.DMA,
        ],
    )
    out = run(dense_flat, idx, table_flat, consts)
    return out.reshape(_BATCH, 1)


# speedup vs baseline: 1.1624x; 1.0098x over previous
import jax
import jax.numpy as jnp
from jax import lax
from jax.experimental import pallas as pl
from jax.experimental.pallas import tpu as pltpu, tpu_sc as plsc

_VOCAB = 1000012
_BATCH = 16384
_N_SPARSE = 26
_D_DENSE = 13

_NC = 2
_NS = 16
_NW = _NC * _NS
_BPW = _BATCH // _NW
_IPW = _BPW * _N_SPARSE
_NBLK = _BPW // 16


def _sc_body(dense_hbm, idx_hbm, table_hbm, wd_hbm, bias_hbm, out_hbm,
             idx_v, vals_v, dense_v, consts_v, out_v, sem):
    wid = lax.axis_index("s") * _NC + lax.axis_index("c")
    base = wid * _BPW

    pltpu.sync_copy(idx_hbm.at[pl.ds(base * _N_SPARSE, _IPW)], idx_v)
    gat = pltpu.async_copy(table_hbm.at[idx_v], vals_v, sem)
    pltpu.sync_copy(dense_hbm.at[pl.ds(base * _D_DENSE, _BPW * _D_DENSE)],
                    dense_v)
    pltpu.sync_copy(wd_hbm, consts_v.at[pl.ds(0, _D_DENSE)])
    pltpu.sync_copy(bias_hbm, consts_v.at[pl.ds(16, 1)])
    gat.wait()

    cv = consts_v[pl.ds(0, 16)]
    bv = consts_v[pl.ds(8, 16)]

    def blk_body(blk, carry):
        rows = lax.iota(jnp.int32, 16) + blk * 16
        acc = jnp.full((16,), bv[8], jnp.float32)
        dbase = rows * _D_DENSE
        for k in range(_D_DENSE):
            acc = acc + plsc.load_gather(dense_v, [dbase + k]) * cv[k]
        vbase = rows * _N_SPARSE
        for f in range(_N_SPARSE):
            acc = acc + plsc.load_gather(vals_v, [vbase + f])
        out_v[pl.ds(blk * 16, 16)] = acc
        return carry

    lax.fori_loop(0, _NBLK, blk_body, 0)
    pltpu.sync_copy(out_v, out_hbm.at[pl.ds(base, _BPW)])


def kernel(dense_input, sparse_input, weight_sparse, weight_dense, bias):
    idx = sparse_input.astype(jnp.int32).reshape(_BATCH * _N_SPARSE)
    dense_flat = dense_input.reshape(_BATCH * _D_DENSE)
    table_flat = weight_sparse.reshape(_VOCAB)
    wd_flat = weight_dense.reshape(_D_DENSE)
    mesh = plsc.VectorSubcoreMesh(core_axis_name="c", subcore_axis_name="s")
    run = pl.kernel(
        _sc_body,
        out_type=jax.ShapeDtypeStruct((_BATCH,), jnp.float32),
        mesh=mesh,
        compiler_params=pltpu.CompilerParams(needs_layout_passes=False),
        scratch_types=[
            pltpu.VMEM((_IPW,), jnp.int32),
            pltpu.VMEM((_IPW,), jnp.float32),
            pltpu.VMEM((_BPW * _D_DENSE,), jnp.float32),
            pltpu.VMEM((24,), jnp.float32),
            pltpu.VMEM((_BPW,), jnp.float32),
            pltpu.SemaphoreType.DMA,
        ],
    )
    out = run(dense_flat, idx, table_flat, wd_flat, bias)
    return out.reshape(_BATCH, 1)


# speedup vs baseline: 2.1777x; 1.8735x over previous
import jax
import jax.numpy as jnp
from jax import lax
from jax.experimental import pallas as pl
from jax.experimental.pallas import tpu as pltpu, tpu_sc as plsc

_VOCAB = 1000012
_VOCAB_PAD = 1000448
_BATCH = 16384
_N_SPARSE = 26
_D_DENSE = 13

_NC = 2
_NS = 16
_NW = _NC * _NS
_BPW = _BATCH // _NW
_NBLK = _BPW // 16


def _sc_body(dense_hbm, sparse_hbm, table_hbm, wd_hbm, bias_hbm, out_hbm,
             idx_v, vals_v, dn_v, consts_v, out_v, sem, dsem):
    wid = lax.axis_index("s") * _NC + lax.axis_index("c")
    base = wid * _BPW

    def sp_row(f, carry):
        pltpu.sync_copy(sparse_hbm.at[f, pl.ds(base, _BPW)],
                        idx_v.at[pl.ds(f * _BPW, _BPW)])
        return carry

    lax.fori_loop(0, _N_SPARSE, sp_row, 0)
    gat = pltpu.async_copy(table_hbm.at[idx_v], vals_v, sem)

    def dn_row(k, carry):
        pltpu.async_copy(dense_hbm.at[k, pl.ds(base, _BPW)],
                         dn_v.at[pl.ds(k * _BPW, _BPW)], dsem)
        return carry

    lax.fori_loop(0, _D_DENSE, dn_row, 0)
    pltpu.sync_copy(wd_hbm, consts_v.at[pl.ds(0, _D_DENSE)])
    pltpu.sync_copy(bias_hbm, consts_v.at[pl.ds(16, 1)])

    def drain_d(k, carry):
        pltpu.make_async_copy(dense_hbm.at[k, pl.ds(base, _BPW)],
                              dn_v.at[pl.ds(k * _BPW, _BPW)], dsem).wait()
        return carry

    lax.fori_loop(0, _D_DENSE, drain_d, 0)
    gat.wait()

    cv = consts_v[pl.ds(0, 16)]
    bv = consts_v[pl.ds(8, 16)]

    def blk_body(blk, carry):
        off = blk * 16
        acc = jnp.full((16,), bv[8], jnp.float32)
        for k in range(_D_DENSE):
            acc = acc + dn_v[pl.ds(k * _BPW + off, 16)] * cv[k]
        for f in range(_N_SPARSE):
            acc = acc + vals_v[pl.ds(f * _BPW + off, 16)]
        out_v[pl.ds(off, 16)] = acc
        return carry

    lax.fori_loop(0, _NBLK, blk_body, 0)
    pltpu.sync_copy(out_v, out_hbm.at[pl.ds(base, _BPW)])


def kernel(dense_input, sparse_input, weight_sparse, weight_dense, bias):
    sparse_t = sparse_input.astype(jnp.int32).T
    dense_t = dense_input.T
    table_flat = jnp.pad(
        weight_sparse, ((0, _VOCAB_PAD - _VOCAB), (0, 0))).reshape(_VOCAB_PAD)
    wd_flat = weight_dense.reshape(_D_DENSE)
    mesh = plsc.VectorSubcoreMesh(core_axis_name="c", subcore_axis_name="s")
    run = pl.kernel(
        _sc_body,
        out_type=jax.ShapeDtypeStruct((_BATCH,), jnp.float32),
        mesh=mesh,
        compiler_params=pltpu.CompilerParams(needs_layout_passes=False),
        scratch_types=[
            pltpu.VMEM((_N_SPARSE * _BPW,), jnp.int32),
            pltpu.VMEM((_N_SPARSE * _BPW,), jnp.float32),
            pltpu.VMEM((_D_DENSE * _BPW,), jnp.float32),
            pltpu.VMEM((24,), jnp.float32),
            pltpu.VMEM((_BPW,), jnp.float32),
            pltpu.SemaphoreType.DMA,
            pltpu.SemaphoreType.DMA,
        ],
    )
    out = run(dense_t, sparse_t, table_flat, wd_flat, bias)
    return out.reshape(_BATCH, 1)


# speedup vs baseline: 2.6021x; 1.1949x over previous
import jax
import jax.numpy as jnp
from jax import lax
from jax.experimental import pallas as pl
from jax.experimental.pallas import tpu as pltpu, tpu_sc as plsc

_VOCAB = 1000012
_VOCAB_PAD = 1000448
_BATCH = 16384
_N_SPARSE = 26
_D_DENSE = 13

_NC = 2
_NS = 16
_NW = _NC * _NS
_BPW = _BATCH // _NW
_NBLK = _BPW // 16


def _sc_body(dense_hbm, sparse_hbm, table_hbm, wd_hbm, bias_hbm, out_hbm,
             idx_v, vals_v, dn_v, consts_v, out_v, sem, dsem, isem):
    wid = lax.axis_index("s") * _NC + lax.axis_index("c")
    base = wid * _BPW

    def sp_row(f, carry):
        pltpu.async_copy(sparse_hbm.at[f, pl.ds(base, _BPW)],
                         idx_v.at[pl.ds(f * _BPW, _BPW)], isem)
        return carry

    lax.fori_loop(0, _N_SPARSE, sp_row, 0)

    def dn_row(k, carry):
        pltpu.async_copy(dense_hbm.at[k, pl.ds(base, _BPW)],
                         dn_v.at[pl.ds(k * _BPW, _BPW)], dsem)
        return carry

    lax.fori_loop(0, _D_DENSE, dn_row, 0)
    pltpu.sync_copy(wd_hbm, consts_v.at[pl.ds(0, _D_DENSE)])
    pltpu.sync_copy(bias_hbm, consts_v.at[pl.ds(16, 1)])

    def drain_i(f, carry):
        pltpu.make_async_copy(sparse_hbm.at[f, pl.ds(base, _BPW)],
                              idx_v.at[pl.ds(f * _BPW, _BPW)], isem).wait()
        return carry

    lax.fori_loop(0, _N_SPARSE, drain_i, 0)

    def gat_row(f, carry):
        pltpu.async_copy(table_hbm.at[idx_v.at[pl.ds(f * _BPW, _BPW)]],
                         vals_v.at[pl.ds(f * _BPW, _BPW)], sem)
        return carry

    lax.fori_loop(0, _N_SPARSE, gat_row, 0)

    def drain_d(k, carry):
        pltpu.make_async_copy(dense_hbm.at[k, pl.ds(base, _BPW)],
                              dn_v.at[pl.ds(k * _BPW, _BPW)], dsem).wait()
        return carry

    lax.fori_loop(0, _D_DENSE, drain_d, 0)

    cv = consts_v[pl.ds(0, 16)]
    bv = consts_v[pl.ds(8, 16)]

    def blk_body(blk, carry):
        off = blk * 16
        acc = jnp.full((16,), bv[8], jnp.float32)
        for k in range(_D_DENSE):
            acc = acc + dn_v[pl.ds(k * _BPW + off, 16)] * cv[k]
        out_v[pl.ds(off, 16)] = acc
        return carry

    lax.fori_loop(0, _NBLK, blk_body, 0)

    def acc_field(f, carry):
        pltpu.make_async_copy(table_hbm.at[idx_v.at[pl.ds(f * _BPW, _BPW)]],
                              vals_v.at[pl.ds(f * _BPW, _BPW)], sem).wait()

        def acc_blk(blk, carry2):
            off = blk * 16
            out_v[pl.ds(off, 16)] = (
                out_v[pl.ds(off, 16)] + vals_v[pl.ds(f * _BPW + off, 16)])
            return carry2

        return lax.fori_loop(0, _NBLK, acc_blk, carry)

    lax.fori_loop(0, _N_SPARSE, acc_field, 0)
    pltpu.sync_copy(out_v, out_hbm.at[pl.ds(base, _BPW)])


def kernel(dense_input, sparse_input, weight_sparse, weight_dense, bias):
    sparse_t = sparse_input.astype(jnp.int32).T
    dense_t = dense_input.T
    table_flat = jnp.pad(
        weight_sparse, ((0, _VOCAB_PAD - _VOCAB), (0, 0))).reshape(_VOCAB_PAD)
    wd_flat = weight_dense.reshape(_D_DENSE)
    mesh = plsc.VectorSubcoreMesh(core_axis_name="c", subcore_axis_name="s")
    run = pl.kernel(
        _sc_body,
        out_type=jax.ShapeDtypeStruct((_BATCH,), jnp.float32),
        mesh=mesh,
        compiler_params=pltpu.CompilerParams(needs_layout_passes=False),
        scratch_types=[
            pltpu.VMEM((_N_SPARSE * _BPW,), jnp.int32),
            pltpu.VMEM((_N_SPARSE * _BPW,), jnp.float32),
            pltpu.VMEM((_D_DENSE * _BPW,), jnp.float32),
            pltpu.VMEM((24,), jnp.float32),
            pltpu.VMEM((_BPW,), jnp.float32),
            pltpu.SemaphoreType.DMA,
            pltpu.SemaphoreType.DMA,
            pltpu.SemaphoreType.DMA,
        ],
    )
    out = run(dense_t, sparse_t, table_flat, wd_flat, bias)
    return out.reshape(_BATCH, 1)


# speedup vs baseline: 2.6537x; 1.0198x over previous
import jax
import jax.numpy as jnp
from jax import lax
from jax.experimental import pallas as pl
from jax.experimental.pallas import tpu as pltpu, tpu_sc as plsc

_VOCAB = 1000012
_VOCAB_PAD = 1000448
_BATCH = 16384
_N_SPARSE = 26
_D_DENSE = 13

_NC = 2
_NS = 16
_NW = _NC * _NS
_BPW = _BATCH // _NW
_NBLK = _BPW // 16


def _sc_body(dense_hbm, sparse_hbm, table_hbm, wd_hbm, bias_hbm, out_hbm,
             idx_v, vals_v, dn_v, consts_v, out_v, sem, dsem, isem):
    wid = lax.axis_index("s") * _NC + lax.axis_index("c")
    base = wid * _BPW

    def sp_row(f, carry):
        pltpu.async_copy(sparse_hbm.at[f, pl.ds(base, _BPW)],
                         idx_v.at[pl.ds(f * _BPW, _BPW)], isem)
        return carry

    lax.fori_loop(0, _N_SPARSE, sp_row, 0)

    def dn_row(k, carry):
        pltpu.async_copy(dense_hbm.at[k, pl.ds(base, _BPW)],
                         dn_v.at[pl.ds(k * _BPW, _BPW)], dsem)
        return carry

    lax.fori_loop(0, _D_DENSE, dn_row, 0)
    pltpu.sync_copy(wd_hbm, consts_v.at[pl.ds(0, _D_DENSE)])
    pltpu.sync_copy(bias_hbm, consts_v.at[pl.ds(16, 1)])

    def gat_row(f, carry):
        pltpu.make_async_copy(sparse_hbm.at[f, pl.ds(base, _BPW)],
                              idx_v.at[pl.ds(f * _BPW, _BPW)], isem).wait()
        pltpu.async_copy(table_hbm.at[idx_v.at[pl.ds(f * _BPW, _BPW)]],
                         vals_v.at[pl.ds(f * _BPW, _BPW)], sem)
        return carry

    lax.fori_loop(0, _N_SPARSE, gat_row, 0)

    def drain_d(k, carry):
        pltpu.make_async_copy(dense_hbm.at[k, pl.ds(base, _BPW)],
                              dn_v.at[pl.ds(k * _BPW, _BPW)], dsem).wait()
        return carry

    lax.fori_loop(0, _D_DENSE, drain_d, 0)

    cv = consts_v[pl.ds(0, 16)]
    bv = consts_v[pl.ds(8, 16)]

    def blk_body(blk, carry):
        off = blk * 16
        acc = jnp.full((16,), bv[8], jnp.float32)
        for k in range(_D_DENSE):
            acc = acc + dn_v[pl.ds(k * _BPW + off, 16)] * cv[k]
        out_v[pl.ds(off, 16)] = acc
        return carry

    lax.fori_loop(0, _NBLK, blk_body, 0)

    def acc_field(f, carry):
        pltpu.make_async_copy(table_hbm.at[idx_v.at[pl.ds(f * _BPW, _BPW)]],
                              vals_v.at[pl.ds(f * _BPW, _BPW)], sem).wait()

        def acc_blk(blk, carry2):
            off = blk * 16
            out_v[pl.ds(off, 16)] = (
                out_v[pl.ds(off, 16)] + vals_v[pl.ds(f * _BPW + off, 16)])
            return carry2

        return lax.fori_loop(0, _NBLK, acc_blk, carry, unroll=8)

    lax.fori_loop(0, _N_SPARSE, acc_field, 0)
    pltpu.sync_copy(out_v, out_hbm.at[pl.ds(base, _BPW)])


def kernel(dense_input, sparse_input, weight_sparse, weight_dense, bias):
    sparse_t = sparse_input.astype(jnp.int32).T
    dense_t = dense_input.T
    table_flat = jnp.pad(
        weight_sparse, ((0, _VOCAB_PAD - _VOCAB), (0, 0))).reshape(_VOCAB_PAD)
    wd_flat = weight_dense.reshape(_D_DENSE)
    mesh = plsc.VectorSubcoreMesh(core_axis_name="c", subcore_axis_name="s")
    run = pl.kernel(
        _sc_body,
        out_type=jax.ShapeDtypeStruct((_BATCH,), jnp.float32),
        mesh=mesh,
        compiler_params=pltpu.CompilerParams(needs_layout_passes=False),
        scratch_types=[
            pltpu.VMEM((_N_SPARSE * _BPW,), jnp.int32),
            pltpu.VMEM((_N_SPARSE * _BPW,), jnp.float32),
            pltpu.VMEM((_D_DENSE * _BPW,), jnp.float32),
            pltpu.VMEM((24,), jnp.float32),
            pltpu.VMEM((_BPW,), jnp.float32),
            pltpu.SemaphoreType.DMA,
            pltpu.SemaphoreType.DMA,
            pltpu.SemaphoreType.DMA,
        ],
    )
    out = run(dense_t, sparse_t, table_flat, wd_flat, bias)
    return out.reshape(_BATCH, 1)


# speedup vs baseline: 2.7685x; 1.0433x over previous
import jax
import jax.numpy as jnp
from jax import lax
from jax.experimental import pallas as pl
from jax.experimental.pallas import tpu as pltpu, tpu_sc as plsc

_VOCAB = 1000012
_VOCAB_PAD = 1000448
_BATCH = 16384
_N_SPARSE = 26
_D_DENSE = 13

_NC = 2
_NS = 16
_NW = _NC * _NS
_BPW = _BATCH // _NW
_NBLK = _BPW // 16
_CHUNKS = ((0, 7), (7, 14), (14, 20), (20, 26))


def _sc_body(dense_hbm, sparse_hbm, table_hbm, wd_hbm, bias_hbm, out_hbm,
             idx_v, vals_v, dn_v, consts_v, out_v, sem, dsem, isem):
    wid = lax.axis_index("s") * _NC + lax.axis_index("c")
    base = wid * _BPW

    def sp_row(f, carry):
        pltpu.async_copy(sparse_hbm.at[f, pl.ds(base, _BPW)],
                         idx_v.at[pl.ds(f * _BPW, _BPW)], isem)
        return carry

    lax.fori_loop(0, _N_SPARSE, sp_row, 0)

    def dn_row(k, carry):
        pltpu.async_copy(dense_hbm.at[k, pl.ds(base, _BPW)],
                         dn_v.at[pl.ds(k * _BPW, _BPW)], dsem)
        return carry

    lax.fori_loop(0, _D_DENSE, dn_row, 0)
    pltpu.sync_copy(wd_hbm, consts_v.at[pl.ds(0, _D_DENSE)])
    pltpu.sync_copy(bias_hbm, consts_v.at[pl.ds(16, 1)])

    for (s, e) in _CHUNKS:
        def drain_i(f, carry):
            pltpu.make_async_copy(sparse_hbm.at[f, pl.ds(base, _BPW)],
                                  idx_v.at[pl.ds(f * _BPW, _BPW)],
                                  isem).wait()
            return carry

        lax.fori_loop(s, e, drain_i, 0)
        n = (e - s) * _BPW
        pltpu.async_copy(table_hbm.at[idx_v.at[pl.ds(s * _BPW, n)]],
                         vals_v.at[pl.ds(s * _BPW, n)], sem)

    def drain_d(k, carry):
        pltpu.make_async_copy(dense_hbm.at[k, pl.ds(base, _BPW)],
                              dn_v.at[pl.ds(k * _BPW, _BPW)], dsem).wait()
        return carry

    lax.fori_loop(0, _D_DENSE, drain_d, 0)

    cv = consts_v[pl.ds(0, 16)]
    bv = consts_v[pl.ds(8, 16)]

    def blk_body(blk, carry):
        off = blk * 16
        acc = jnp.full((16,), bv[8], jnp.float32)
        for k in range(_D_DENSE):
            acc = acc + dn_v[pl.ds(k * _BPW + off, 16)] * cv[k]
        out_v[pl.ds(off, 16)] = acc
        return carry

    lax.fori_loop(0, _NBLK, blk_body, 0)

    for (s, e) in _CHUNKS:
        n = (e - s) * _BPW
        pltpu.make_async_copy(table_hbm.at[idx_v.at[pl.ds(s * _BPW, n)]],
                              vals_v.at[pl.ds(s * _BPW, n)], sem).wait()

        def acc_blk(blk, carry2):
            off = blk * 16
            acc = out_v[pl.ds(off, 16)]
            for f in range(s, e):
                acc = acc + vals_v[pl.ds(f * _BPW + off, 16)]
            out_v[pl.ds(off, 16)] = acc
            return carry2

        lax.fori_loop(0, _NBLK, acc_blk, 0, unroll=4)
    pltpu.sync_copy(out_v, out_hbm.at[pl.ds(base, _BPW)])


def kernel(dense_input, sparse_input, weight_sparse, weight_dense, bias):
    sparse_t = sparse_input.astype(jnp.int32).T
    dense_t = dense_input.T
    table_flat = jnp.pad(
        weight_sparse, ((0, _VOCAB_PAD - _VOCAB), (0, 0))).reshape(_VOCAB_PAD)
    wd_flat = weight_dense.reshape(_D_DENSE)
    mesh = plsc.VectorSubcoreMesh(core_axis_name="c", subcore_axis_name="s")
    run = pl.kernel(
        _sc_body,
        out_type=jax.ShapeDtypeStruct((_BATCH,), jnp.float32),
        mesh=mesh,
        compiler_params=pltpu.CompilerParams(needs_layout_passes=False),
        scratch_types=[
            pltpu.VMEM((_N_SPARSE * _BPW,), jnp.int32),
            pltpu.VMEM((_N_SPARSE * _BPW,), jnp.float32),
            pltpu.VMEM((_D_DENSE * _BPW,), jnp.float32),
            pltpu.VMEM((24,), jnp.float32),
            pltpu.VMEM((_BPW,), jnp.float32),
            pltpu.SemaphoreType.DMA,
            pltpu.SemaphoreType.DMA,
            pltpu.SemaphoreType.DMA,
        ],
    )
    out = run(dense_t, sparse_t, table_flat, wd_flat, bias)
    return out.reshape(_BATCH, 1)
